# baseline (speedup 1.0000x reference)
import functools

import jax
import jax.numpy as jnp
from jax.experimental import pallas as pl
from jax.experimental.pallas import tpu as pltpu

TEMPERATURE = 1.0
EPS = 1e-20
B, V = 128, 100000

ROWS = 8
NROW = B // ROWS
VPAD = 100096
HOT_W = 4352
NHOT = VPAD // HOT_W

INT_MAX = 2**31 - 1


def _gumbel_argmax_kernel(l_ref, u_ref, hot_ref, idx_ref):
    r = pl.program_id(0)

    @pl.when(r < NROW)
    def _argmax():
        g = -jnp.log(-jnp.log(u_ref[...] + EPS) + EPS)
        s = l_ref[...] + g
        bmax = jnp.max(s, axis=1, keepdims=True)
        col = jax.lax.broadcasted_iota(jnp.int32, s.shape, 1)
        bidx = jnp.min(jnp.where(s == bmax, col, INT_MAX), axis=1,
                       keepdims=True)
        idx_ref[pl.ds(r * ROWS, ROWS), :] = bidx

    @pl.when(r == NROW)
    def _one_hot():
        idx = idx_ref[...]
        for c in range(NHOT):
            col = (jax.lax.broadcasted_iota(jnp.int32, (B, HOT_W), 1)
                   + c * HOT_W)
            anyhot = jnp.any(col == idx, axis=0, keepdims=True)
            hot_ref[:, pl.ds(c * HOT_W, HOT_W)] = anyhot.astype(jnp.float32)


@functools.partial(jax.jit, static_argnames=("interpret",))
def kernel(logits, gumbel_u, interpret=False):
    hot = pl.pallas_call(
        _gumbel_argmax_kernel,
        grid=(NROW + 1,),
        in_specs=[
            pl.BlockSpec((ROWS, V), lambda r: (jnp.minimum(r, NROW - 1), 0)),
            pl.BlockSpec((ROWS, V), lambda r: (jnp.minimum(r, NROW - 1), 0)),
        ],
        out_specs=pl.BlockSpec((1, VPAD), lambda r: (0, 0)),
        out_shape=jax.ShapeDtypeStruct((1, VPAD), jnp.float32),
        scratch_shapes=[pltpu.VMEM((B, 1), jnp.int32)],
        interpret=interpret,
    )(logits, gumbel_u)

    return jax.lax.dynamic_update_slice(
        jnp.zeros((B, V), jnp.float32), hot[:, :V], (0, 0))
